# Initial kernel scaffold; baseline (speedup 1.0000x reference)
import jax
import jax.numpy as jnp
from jax.experimental import pallas as pl


def kernel(edge_index, node_attr, edge_attr, W1, b1, g1, be1, W2, b2, g2, be2, W3, b3, g3, be3):
    raise NotImplementedError("write your pallas kernel here")


# speedup vs baseline: 16.9311x; 16.9311x over previous
import functools

import jax
import jax.numpy as jnp
from jax import lax
from jax.experimental import pallas as pl
from jax.experimental.pallas import tpu as pltpu
from jax.experimental.pallas import tpu_sc as plsc

N = 10000
D = 128
E = 320000
NC = 2
NS = 16
NW = NC * NS
EPW = E // NW
CHUNK = 80
NCH = EPW // CHUNK
RSTRIDE = 624
RSPAN = 640
NPAD = 10240
PPT = NPAD // NS
EPS = 1e-5

@functools.cache
def _sc_kernels():
    mesh = plsc.VectorSubcoreMesh(core_axis_name="c", subcore_axis_name="s")

    @functools.partial(
        pl.kernel,
        mesh=mesh,
        out_type=jax.ShapeDtypeStruct((NC * NPAD,), jnp.float32),
        scratch_types=[
            pltpu.VMEM((NCH, CHUNK), jnp.int32),
            pltpu.VMEM((PPT,), jnp.float32),
            pltpu.VMEM_SHARED((NPAD,), jnp.float32),
        ],
    )
    def deg_kernel(dst_hbm, out_hbm, dst_v, zv, deg_sh):
        c = lax.axis_index("c")
        s = lax.axis_index("s")
        wid = s * NC + c

        def zero_body(i, _):
            zv[pl.ds(i * 16, 16)] = jnp.zeros((16,), jnp.float32)
            return ()

        lax.fori_loop(0, PPT // 16, zero_body, ())
        pltpu.sync_copy(zv, deg_sh.at[pl.ds(s * PPT, PPT)])
        pltpu.sync_copy(dst_hbm.at[wid], dst_v)

        def ones_body(i, _):
            zv[pl.ds(i * 16, 16)] = jnp.ones((16,), jnp.float32)
            return ()

        lax.fori_loop(0, CHUNK // 16, ones_body, ())
        plsc.subcore_barrier()

        def body(j, _):
            pltpu.sync_copy(zv.at[pl.ds(0, CHUNK)],
                            deg_sh.at[dst_v.at[j]], add=True)
            return ()

        lax.fori_loop(0, NCH, body, ())
        plsc.subcore_barrier()
        pltpu.sync_copy(deg_sh.at[pl.ds(s * PPT, PPT)],
                        out_hbm.at[pl.ds(c * NPAD + s * PPT, PPT)])

    @functools.partial(
        pl.kernel,
        mesh=mesh,
        out_type=jax.ShapeDtypeStruct((NC * N, D), jnp.float32),
        scratch_types=[
            pltpu.VMEM((NCH, CHUNK), jnp.int32),
            pltpu.VMEM((NCH, CHUNK), jnp.int32),
            pltpu.VMEM((CHUNK, D), jnp.float32),
            pltpu.VMEM_SHARED((N, D), jnp.float32),
            pltpu.SemaphoreType.DMA,
        ],
    )
    def edge_kernel(g_hbm, src_hbm, dst_hbm, out_hbm, src_v, dst_v, rows_v,
                    acc_sh, sem):
        c = lax.axis_index("c")
        s = lax.axis_index("s")
        wid = s * NC + c
        pltpu.sync_copy(g_hbm.at[pl.ds(s * RSTRIDE, RSPAN)],
                        acc_sh.at[pl.ds(s * RSTRIDE, RSPAN)])
        pltpu.sync_copy(src_hbm.at[wid], src_v)
        pltpu.sync_copy(dst_hbm.at[wid], dst_v)
        plsc.subcore_barrier()

        def body(j, _):
            pltpu.async_copy(g_hbm.at[src_v.at[j]], rows_v, sem).wait()
            pltpu.sync_copy(rows_v, acc_sh.at[dst_v.at[j]], add=True)
            return ()

        lax.fori_loop(0, NCH, body, ())
        plsc.subcore_barrier()
        pltpu.sync_copy(acc_sh.at[pl.ds(s * RSTRIDE, RSPAN)],
                        out_hbm.at[pl.ds(c * N + s * RSTRIDE, RSPAN)])

    return deg_kernel, edge_kernel


def _prep_body(pT_ref, x_ref, w_ref, dis_ref, g_ref):
    ones = jnp.ones((NC, D), jnp.float32)
    deg = jnp.dot(pT_ref[...], ones, preferred_element_type=jnp.float32) + 1.0
    dis = lax.rsqrt(deg)
    h = jnp.dot(x_ref[...], w_ref[...], preferred_element_type=jnp.float32)
    dis_ref[...] = dis
    g_ref[...] = h * dis


_prep_call = pl.pallas_call(
    _prep_body,
    out_shape=[
        jax.ShapeDtypeStruct((N, D), jnp.float32),
        jax.ShapeDtypeStruct((N, D), jnp.float32),
    ],
)


def _combine_body(acc_ref, g_ref, dis_ref, b_ref, gam_ref, bet_ref, w_ref,
                  x_ref, g_out_ref):
    acc = acc_ref[0:N, :] + acc_ref[N:2 * N, :]
    dis = dis_ref[...]
    pre = dis * (acc - g_ref[...]) + b_ref[...]
    y = jnp.maximum(pre, 0.0)
    m = jnp.mean(y, axis=0, keepdims=True)
    d = y - m
    v = jnp.mean(d * d, axis=0, keepdims=True)
    xn = gam_ref[...] * d * lax.rsqrt(v + EPS) + bet_ref[...]
    x_ref[...] = xn
    g_out_ref[...] = jnp.dot(xn, w_ref[...],
                             preferred_element_type=jnp.float32) * dis


_combine_call = pl.pallas_call(
    _combine_body,
    out_shape=[
        jax.ShapeDtypeStruct((N, D), jnp.float32),
        jax.ShapeDtypeStruct((N, D), jnp.float32),
    ],
)


def _combine_final_body(acc_ref, g_ref, dis_ref, b_ref, gam_ref, bet_ref,
                        x_ref):
    acc = acc_ref[0:N, :] + acc_ref[N:2 * N, :]
    pre = dis_ref[...] * (acc - g_ref[...]) + b_ref[...]
    y = jnp.maximum(pre, 0.0)
    m = jnp.mean(y, axis=0, keepdims=True)
    d = y - m
    v = jnp.mean(d * d, axis=0, keepdims=True)
    x_ref[...] = gam_ref[...] * d * lax.rsqrt(v + EPS) + bet_ref[...]


_combine_final_call = pl.pallas_call(
    _combine_final_body,
    out_shape=jax.ShapeDtypeStruct((N, D), jnp.float32),
)


def kernel(edge_index, node_attr, edge_attr,
           W1, b1, g1, be1, W2, b2, g2, be2, W3, b3, g3, be3):
    del edge_attr
    src = edge_index[0].astype(jnp.int32).reshape(NW, NCH, CHUNK)
    dst = edge_index[1].astype(jnp.int32).reshape(NW, NCH, CHUNK)
    b1r = b1.reshape(1, D); b2r = b2.reshape(1, D); b3r = b3.reshape(1, D)
    g1r = g1.reshape(1, D); g2r = g2.reshape(1, D); g3r = g3.reshape(1, D)
    be1r = be1.reshape(1, D); be2r = be2.reshape(1, D); be3r = be3.reshape(1, D)

    deg_kernel, edge_kernel = _sc_kernels()
    partials = deg_kernel(dst)
    pT = partials.reshape(NC, NPAD)[:, :N].T
    dis, gg = _prep_call(pT, node_attr, W1)

    acc = edge_kernel(gg, src, dst)
    x2, gg2 = _combine_call(acc, gg, dis, b1r, g1r, be1r, W2)
    del x2
    acc2 = edge_kernel(gg2, src, dst)
    x3, gg3 = _combine_call(acc2, gg2, dis, b2r, g2r, be2r, W3)
    del x3
    acc3 = edge_kernel(gg3, src, dst)
    return _combine_final_call(acc3, gg3, dis, b3r, g3r, be3r)


# speedup vs baseline: 21.8942x; 1.2931x over previous
import functools

import jax
import jax.numpy as jnp
from jax import lax
from jax.experimental import pallas as pl
from jax.experimental.pallas import tpu as pltpu
from jax.experimental.pallas import tpu_sc as plsc

N = 10000
D = 128
E = 320000
NC = 2
NS = 16
NW = NC * NS
EPW = E // NW
CHUNK = 80
NCH = EPW // CHUNK
RSTRIDE = 624
RSPAN = 640
NPAD = 10240
PPT = NPAD // NS
EPS = 1e-5

@functools.cache
def _sc_kernels():
    mesh = plsc.VectorSubcoreMesh(core_axis_name="c", subcore_axis_name="s")

    @functools.partial(
        pl.kernel,
        mesh=mesh,
        out_type=jax.ShapeDtypeStruct((NC * NPAD,), jnp.float32),
        scratch_types=[
            pltpu.VMEM((NCH, CHUNK), jnp.int32),
            pltpu.VMEM((PPT,), jnp.float32),
            pltpu.VMEM_SHARED((NPAD,), jnp.float32),
        ],
    )
    def deg_kernel(dst_hbm, out_hbm, dst_v, zv, deg_sh):
        c = lax.axis_index("c")
        s = lax.axis_index("s")
        wid = s * NC + c

        def zero_body(i, _):
            zv[pl.ds(i * 16, 16)] = jnp.zeros((16,), jnp.float32)
            return ()

        lax.fori_loop(0, PPT // 16, zero_body, ())
        pltpu.sync_copy(zv, deg_sh.at[pl.ds(s * PPT, PPT)])
        pltpu.sync_copy(dst_hbm.at[wid], dst_v)

        def ones_body(i, _):
            zv[pl.ds(i * 16, 16)] = jnp.ones((16,), jnp.float32)
            return ()

        lax.fori_loop(0, CHUNK // 16, ones_body, ())
        plsc.subcore_barrier()

        def body(j, _):
            pltpu.sync_copy(zv.at[pl.ds(0, CHUNK)],
                            deg_sh.at[dst_v.at[j]], add=True)
            return ()

        lax.fori_loop(0, NCH, body, ())
        plsc.subcore_barrier()
        pltpu.sync_copy(deg_sh.at[pl.ds(s * PPT, PPT)],
                        out_hbm.at[pl.ds(c * NPAD + s * PPT, PPT)])

    NBUF = 2
    NGRP = (NCH - 1) // NBUF

    @functools.partial(
        pl.kernel,
        mesh=mesh,
        out_type=jax.ShapeDtypeStruct((NC * N, D), jnp.float32),
        scratch_types=[
            pltpu.VMEM((EPW,), jnp.int32),
            pltpu.VMEM((NCH, CHUNK), jnp.int32),
            pltpu.VMEM((NBUF, CHUNK, D), jnp.float32),
            pltpu.VMEM_SHARED((N, D), jnp.float32),
            pltpu.SemaphoreType.DMA((NBUF,)),
            pltpu.SemaphoreType.DMA((NBUF,)),
            pltpu.SemaphoreType.DMA,
        ],
    )
    def edge_kernel(g_hbm, src_hbm, dst_hbm, out_hbm, src_v, dst_v, rows_b,
                    acc_sh, gsem, ssem, isem):
        rows_v = [rows_b.at[b] for b in range(NBUF)]
        c = lax.axis_index("c")
        s = lax.axis_index("s")
        wid = s * NC + c
        init_cp = pltpu.async_copy(
            g_hbm.at[pl.ds(s * RSTRIDE, RSPAN)],
            acc_sh.at[pl.ds(s * RSTRIDE, RSPAN)], isem)
        src_cp = pltpu.async_copy(src_hbm.at[wid], src_v, gsem.at[0])
        dst_cp = pltpu.async_copy(dst_hbm.at[wid], dst_v, gsem.at[1])
        src_cp.wait()
        dst_cp.wait()
        init_cp.wait()
        plsc.subcore_barrier()

        def sidx(j):
            return src_v.at[pl.ds(j * CHUNK, CHUNK)]

        pltpu.async_copy(g_hbm.at[sidx(0)], rows_v[0], gsem.at[0])
        pltpu.async_copy(g_hbm.at[sidx(1)], rows_v[1], gsem.at[1])

        def group(g, _):
            j0 = 2 * g
            j1 = j0 + 1
            pltpu.make_async_copy(
                g_hbm.at[sidx(j0)], rows_v[0], gsem.at[0]).wait()
            pltpu.async_copy(rows_v[0], acc_sh.at[dst_v.at[j0]], ssem.at[0],
                             add=True)
            pltpu.make_async_copy(
                g_hbm.at[sidx(j1)], rows_v[1], gsem.at[1]).wait()
            pltpu.async_copy(rows_v[1], acc_sh.at[dst_v.at[j1]], ssem.at[1],
                             add=True)
            pltpu.make_async_copy(
                rows_v[0], acc_sh.at[dst_v.at[j0]], ssem.at[0]).wait()
            pltpu.async_copy(g_hbm.at[sidx(j0 + 2)], rows_v[0], gsem.at[0])
            pltpu.make_async_copy(
                rows_v[1], acc_sh.at[dst_v.at[j1]], ssem.at[1]).wait()

            @pl.when(g < NGRP - 1)
            def _():
                pltpu.async_copy(g_hbm.at[sidx(j1 + 2)], rows_v[1],
                                 gsem.at[1])

            return ()

        lax.fori_loop(0, NGRP, group, ())
        j_last = NGRP * NBUF
        pltpu.make_async_copy(
            g_hbm.at[sidx(j_last)], rows_v[0], gsem.at[0]).wait()
        pltpu.sync_copy(rows_v[0], acc_sh.at[dst_v.at[j_last]], add=True)
        plsc.subcore_barrier()
        pltpu.sync_copy(acc_sh.at[pl.ds(s * RSTRIDE, RSPAN)],
                        out_hbm.at[pl.ds(c * N + s * RSTRIDE, RSPAN)])

    return deg_kernel, edge_kernel


def _prep_body(pT_ref, x_ref, w_ref, dis_ref, g_ref):
    ones = jnp.ones((NC, D), jnp.float32)
    deg = jnp.dot(pT_ref[...], ones, preferred_element_type=jnp.float32) + 1.0
    dis = lax.rsqrt(deg)
    h = jnp.dot(x_ref[...], w_ref[...], preferred_element_type=jnp.float32)
    dis_ref[...] = dis
    g_ref[...] = h * dis


_prep_call = pl.pallas_call(
    _prep_body,
    out_shape=[
        jax.ShapeDtypeStruct((N, D), jnp.float32),
        jax.ShapeDtypeStruct((N, D), jnp.float32),
    ],
)


def _combine_body(acc_ref, g_ref, dis_ref, b_ref, gam_ref, bet_ref, w_ref,
                  x_ref, g_out_ref):
    acc = acc_ref[0:N, :] + acc_ref[N:2 * N, :]
    dis = dis_ref[...]
    pre = dis * (acc - g_ref[...]) + b_ref[...]
    y = jnp.maximum(pre, 0.0)
    m = jnp.mean(y, axis=0, keepdims=True)
    d = y - m
    v = jnp.mean(d * d, axis=0, keepdims=True)
    xn = gam_ref[...] * d * lax.rsqrt(v + EPS) + bet_ref[...]
    x_ref[...] = xn
    g_out_ref[...] = jnp.dot(xn, w_ref[...],
                             preferred_element_type=jnp.float32) * dis


_combine_call = pl.pallas_call(
    _combine_body,
    out_shape=[
        jax.ShapeDtypeStruct((N, D), jnp.float32),
        jax.ShapeDtypeStruct((N, D), jnp.float32),
    ],
)


def _combine_final_body(acc_ref, g_ref, dis_ref, b_ref, gam_ref, bet_ref,
                        x_ref):
    acc = acc_ref[0:N, :] + acc_ref[N:2 * N, :]
    pre = dis_ref[...] * (acc - g_ref[...]) + b_ref[...]
    y = jnp.maximum(pre, 0.0)
    m = jnp.mean(y, axis=0, keepdims=True)
    d = y - m
    v = jnp.mean(d * d, axis=0, keepdims=True)
    x_ref[...] = gam_ref[...] * d * lax.rsqrt(v + EPS) + bet_ref[...]


_combine_final_call = pl.pallas_call(
    _combine_final_body,
    out_shape=jax.ShapeDtypeStruct((N, D), jnp.float32),
)


def kernel(edge_index, node_attr, edge_attr,
           W1, b1, g1, be1, W2, b2, g2, be2, W3, b3, g3, be3):
    del edge_attr
    src = edge_index[0].astype(jnp.int32).reshape(NW, EPW)
    dst = edge_index[1].astype(jnp.int32).reshape(NW, NCH, CHUNK)
    b1r = b1.reshape(1, D); b2r = b2.reshape(1, D); b3r = b3.reshape(1, D)
    g1r = g1.reshape(1, D); g2r = g2.reshape(1, D); g3r = g3.reshape(1, D)
    be1r = be1.reshape(1, D); be2r = be2.reshape(1, D); be3r = be3.reshape(1, D)

    deg_kernel, edge_kernel = _sc_kernels()
    partials = deg_kernel(dst)
    pT = partials.reshape(NC, NPAD)[:, :N].T
    dis, gg = _prep_call(pT, node_attr, W1)

    acc = edge_kernel(gg, src, dst)
    x2, gg2 = _combine_call(acc, gg, dis, b1r, g1r, be1r, W2)
    del x2
    acc2 = edge_kernel(gg2, src, dst)
    x3, gg3 = _combine_call(acc2, gg2, dis, b2r, g2r, be2r, W3)
    del x3
    acc3 = edge_kernel(gg3, src, dst)
    return _combine_final_call(acc3, gg3, dis, b3r, g3r, be3r)


# speedup vs baseline: 21.9116x; 1.0008x over previous
import functools

import jax
import jax.numpy as jnp
from jax import lax
from jax.experimental import pallas as pl
from jax.experimental.pallas import tpu as pltpu
from jax.experimental.pallas import tpu_sc as plsc

N = 10000
D = 128
E = 320000
NC = 2
NS = 16
NW = NC * NS
EPW = E // NW
CHUNK = 80
NCH = EPW // CHUNK
RSTRIDE = 624
RSPAN = 640
NPAD = 10240
PPT = NPAD // NS
EPS = 1e-5

@functools.cache
def _sc_kernels():
    mesh = plsc.VectorSubcoreMesh(core_axis_name="c", subcore_axis_name="s")

    @functools.partial(
        pl.kernel,
        mesh=mesh,
        out_type=jax.ShapeDtypeStruct((NC * NPAD,), jnp.float32),
        scratch_types=[
            pltpu.VMEM((NCH, CHUNK), jnp.int32),
            pltpu.VMEM((PPT,), jnp.float32),
            pltpu.VMEM_SHARED((NPAD,), jnp.float32),
        ],
    )
    def deg_kernel(dst_hbm, out_hbm, dst_v, zv, deg_sh):
        c = lax.axis_index("c")
        s = lax.axis_index("s")
        wid = s * NC + c

        def zero_body(i, _):
            zv[pl.ds(i * 16, 16)] = jnp.zeros((16,), jnp.float32)
            return ()

        lax.fori_loop(0, PPT // 16, zero_body, ())
        pltpu.sync_copy(zv, deg_sh.at[pl.ds(s * PPT, PPT)])
        pltpu.sync_copy(dst_hbm.at[wid], dst_v)

        def ones_body(i, _):
            zv[pl.ds(i * 16, 16)] = jnp.ones((16,), jnp.float32)
            return ()

        lax.fori_loop(0, CHUNK // 16, ones_body, ())
        plsc.subcore_barrier()

        def body(j, _):
            pltpu.sync_copy(zv.at[pl.ds(0, CHUNK)],
                            deg_sh.at[dst_v.at[j]], add=True)
            return ()

        lax.fori_loop(0, NCH, body, ())
        plsc.subcore_barrier()
        pltpu.sync_copy(deg_sh.at[pl.ds(s * PPT, PPT)],
                        out_hbm.at[pl.ds(c * NPAD + s * PPT, PPT)])

    NBUF = 2
    NGRP = (NCH - 1) // NBUF

    @functools.partial(
        pl.kernel,
        mesh=mesh,
        out_type=jax.ShapeDtypeStruct((NC * N, D), jnp.float32),
        scratch_types=[
            pltpu.VMEM((EPW,), jnp.int32),
            pltpu.VMEM((NCH, CHUNK), jnp.int32),
            pltpu.VMEM((NBUF, CHUNK, D), jnp.float32),
            pltpu.VMEM_SHARED((N, D), jnp.float32),
            pltpu.SemaphoreType.DMA((NBUF,)),
            pltpu.SemaphoreType.DMA((NBUF,)),
            pltpu.SemaphoreType.DMA,
        ],
    )
    def edge_kernel(g_hbm, src_hbm, dst_hbm, out_hbm, src_v, dst_v, rows_b,
                    acc_sh, gsem, ssem, isem):
        rows_v = [rows_b.at[b] for b in range(NBUF)]
        c = lax.axis_index("c")
        s = lax.axis_index("s")
        wid = s * NC + c
        init_cp = pltpu.async_copy(
            g_hbm.at[pl.ds(s * RSTRIDE, RSPAN)],
            acc_sh.at[pl.ds(s * RSTRIDE, RSPAN)], isem)
        src_cp = pltpu.async_copy(src_hbm.at[wid], src_v, gsem.at[0])
        dst_cp = pltpu.async_copy(dst_hbm.at[wid], dst_v, gsem.at[1])
        src_cp.wait()
        dst_cp.wait()
        init_cp.wait()
        plsc.subcore_barrier()

        def sidx(j):
            return src_v.at[pl.ds(j * CHUNK, CHUNK)]

        pltpu.async_copy(g_hbm.at[sidx(0)], rows_v[0], gsem.at[0])
        pltpu.async_copy(g_hbm.at[sidx(1)], rows_v[1], gsem.at[1])

        def group(g, _):
            j0 = 2 * g
            j1 = j0 + 1
            pltpu.make_async_copy(
                g_hbm.at[sidx(j0)], rows_v[0], gsem.at[0]).wait()
            pltpu.async_copy(rows_v[0], acc_sh.at[dst_v.at[j0]], ssem.at[0],
                             add=True)
            pltpu.make_async_copy(
                g_hbm.at[sidx(j1)], rows_v[1], gsem.at[1]).wait()
            pltpu.async_copy(rows_v[1], acc_sh.at[dst_v.at[j1]], ssem.at[1],
                             add=True)
            pltpu.make_async_copy(
                rows_v[0], acc_sh.at[dst_v.at[j0]], ssem.at[0]).wait()
            pltpu.async_copy(g_hbm.at[sidx(j0 + 2)], rows_v[0], gsem.at[0])
            pltpu.make_async_copy(
                rows_v[1], acc_sh.at[dst_v.at[j1]], ssem.at[1]).wait()

            @pl.when(g < NGRP - 1)
            def _():
                pltpu.async_copy(g_hbm.at[sidx(j1 + 2)], rows_v[1],
                                 gsem.at[1])

            return ()

        lax.fori_loop(0, NGRP, group, ())
        j_last = NGRP * NBUF
        pltpu.make_async_copy(
            g_hbm.at[sidx(j_last)], rows_v[0], gsem.at[0]).wait()
        pltpu.sync_copy(rows_v[0], acc_sh.at[dst_v.at[j_last]], add=True)
        plsc.subcore_barrier()
        pltpu.sync_copy(acc_sh.at[pl.ds(s * RSTRIDE, RSPAN)],
                        out_hbm.at[pl.ds(c * N + s * RSTRIDE, RSPAN)])

    return deg_kernel, edge_kernel


def _matmul_body(x_ref, w_ref, h_ref):
    h_ref[...] = jnp.dot(x_ref[...], w_ref[...],
                         preferred_element_type=jnp.float32)


_matmul_call = pl.pallas_call(
    _matmul_body,
    out_shape=jax.ShapeDtypeStruct((N, D), jnp.float32),
)


def _scale_body(pT_ref, h_ref, dis_ref, g_ref):
    ones = jnp.ones((NC, D), jnp.float32)
    deg = jnp.dot(pT_ref[...], ones, preferred_element_type=jnp.float32) + 1.0
    dis = lax.rsqrt(deg)
    dis_ref[...] = dis
    g_ref[...] = h_ref[...] * dis


_scale_call = pl.pallas_call(
    _scale_body,
    out_shape=[
        jax.ShapeDtypeStruct((N, D), jnp.float32),
        jax.ShapeDtypeStruct((N, D), jnp.float32),
    ],
)


def _combine_body(acc_ref, g_ref, dis_ref, b_ref, gam_ref, bet_ref, w_ref,
                  x_ref, g_out_ref):
    acc = acc_ref[0:N, :] + acc_ref[N:2 * N, :]
    dis = dis_ref[...]
    pre = dis * (acc - g_ref[...]) + b_ref[...]
    y = jnp.maximum(pre, 0.0)
    m = jnp.mean(y, axis=0, keepdims=True)
    d = y - m
    v = jnp.mean(d * d, axis=0, keepdims=True)
    xn = gam_ref[...] * d * lax.rsqrt(v + EPS) + bet_ref[...]
    x_ref[...] = xn
    g_out_ref[...] = jnp.dot(xn, w_ref[...],
                             preferred_element_type=jnp.float32) * dis


_combine_call = pl.pallas_call(
    _combine_body,
    out_shape=[
        jax.ShapeDtypeStruct((N, D), jnp.float32),
        jax.ShapeDtypeStruct((N, D), jnp.float32),
    ],
)


def _combine_final_body(acc_ref, g_ref, dis_ref, b_ref, gam_ref, bet_ref,
                        x_ref):
    acc = acc_ref[0:N, :] + acc_ref[N:2 * N, :]
    pre = dis_ref[...] * (acc - g_ref[...]) + b_ref[...]
    y = jnp.maximum(pre, 0.0)
    m = jnp.mean(y, axis=0, keepdims=True)
    d = y - m
    v = jnp.mean(d * d, axis=0, keepdims=True)
    x_ref[...] = gam_ref[...] * d * lax.rsqrt(v + EPS) + bet_ref[...]


_combine_final_call = pl.pallas_call(
    _combine_final_body,
    out_shape=jax.ShapeDtypeStruct((N, D), jnp.float32),
)


def kernel(edge_index, node_attr, edge_attr,
           W1, b1, g1, be1, W2, b2, g2, be2, W3, b3, g3, be3):
    del edge_attr
    src = edge_index[0].astype(jnp.int32).reshape(NW, EPW)
    dst = edge_index[1].astype(jnp.int32).reshape(NW, NCH, CHUNK)
    b1r = b1.reshape(1, D); b2r = b2.reshape(1, D); b3r = b3.reshape(1, D)
    g1r = g1.reshape(1, D); g2r = g2.reshape(1, D); g3r = g3.reshape(1, D)
    be1r = be1.reshape(1, D); be2r = be2.reshape(1, D); be3r = be3.reshape(1, D)

    deg_kernel, edge_kernel = _sc_kernels()
    partials = deg_kernel(dst)
    h1 = _matmul_call(node_attr, W1)
    pT = partials.reshape(NC, NPAD)[:, :N].T
    dis, gg = _scale_call(pT, h1)

    acc = edge_kernel(gg, src, dst)
    x2, gg2 = _combine_call(acc, gg, dis, b1r, g1r, be1r, W2)
    del x2
    acc2 = edge_kernel(gg2, src, dst)
    x3, gg3 = _combine_call(acc2, gg2, dis, b2r, g2r, be2r, W3)
    del x3
    acc3 = edge_kernel(gg3, src, dst)
    return _combine_final_call(acc3, gg3, dis, b3r, g3r, be3r)


# speedup vs baseline: 26.1110x; 1.1917x over previous
import functools

import jax
import jax.numpy as jnp
from jax import lax
from jax.experimental import pallas as pl
from jax.experimental.pallas import tpu as pltpu
from jax.experimental.pallas import tpu_sc as plsc

N = 10000
D = 128
E = 320000
NC = 2
NS = 16
NW = NC * NS
EPW = E // NW
CHUNK = 80
NCH = EPW // CHUNK
RSTRIDE = 624
RSPAN = 640
NPAD = 10240
PPT = NPAD // NS
EPS = 1e-5

@functools.cache
def _sc_kernels():
    mesh = plsc.VectorSubcoreMesh(core_axis_name="c", subcore_axis_name="s")

    @functools.partial(
        pl.kernel,
        mesh=mesh,
        out_type=jax.ShapeDtypeStruct((NC * NPAD,), jnp.float32),
        scratch_types=[
            pltpu.VMEM((NCH, CHUNK), jnp.int32),
            pltpu.VMEM((PPT,), jnp.float32),
            pltpu.VMEM_SHARED((NPAD,), jnp.float32),
        ],
    )
    def deg_kernel(dst_hbm, out_hbm, dst_v, zv, deg_sh):
        c = lax.axis_index("c")
        s = lax.axis_index("s")
        wid = s * NC + c

        def zero_body(i, _):
            zv[pl.ds(i * 16, 16)] = jnp.zeros((16,), jnp.float32)
            return ()

        lax.fori_loop(0, PPT // 16, zero_body, ())
        pltpu.sync_copy(zv, deg_sh.at[pl.ds(s * PPT, PPT)])
        pltpu.sync_copy(dst_hbm.at[wid], dst_v)

        def ones_body(i, _):
            zv[pl.ds(i * 16, 16)] = jnp.ones((16,), jnp.float32)
            return ()

        lax.fori_loop(0, CHUNK // 16, ones_body, ())
        plsc.subcore_barrier()

        def body(j, _):
            pltpu.sync_copy(zv.at[pl.ds(0, CHUNK)],
                            deg_sh.at[dst_v.at[j]], add=True)
            return ()

        lax.fori_loop(0, NCH, body, ())
        plsc.subcore_barrier()
        pltpu.sync_copy(deg_sh.at[pl.ds(s * PPT, PPT)],
                        out_hbm.at[pl.ds(c * NPAD + s * PPT, PPT)])

    NBUF = 3
    DSTG = 64
    NGA = 21
    NGB = 20
    DOFF_B = 64

    @functools.partial(
        pl.kernel,
        mesh=mesh,
        out_type=jax.ShapeDtypeStruct((NC * N, D), jnp.float32),
        scratch_types=[
            pltpu.VMEM((EPW,), jnp.int32),
            pltpu.VMEM((DSTG, CHUNK), jnp.int32),
            pltpu.VMEM((NBUF, CHUNK, D), jnp.float32),
            pltpu.VMEM_SHARED((N, D), jnp.float32),
            pltpu.SemaphoreType.DMA((NBUF,)),
            pltpu.SemaphoreType.DMA((NBUF,)),
            pltpu.SemaphoreType.DMA,
        ],
    )
    def edge_kernel(g_hbm, src_hbm, dst_hbm, out_hbm, src_v, dst_v, rows_b,
                    acc_sh, gsem, ssem, isem):
        rows_v = [rows_b.at[b] for b in range(NBUF)]
        c = lax.axis_index("c")
        s = lax.axis_index("s")
        wid = s * NC + c
        init_cp = pltpu.async_copy(
            g_hbm.at[pl.ds(s * RSTRIDE, RSPAN)],
            acc_sh.at[pl.ds(s * RSTRIDE, RSPAN)], isem)
        src_cp = pltpu.async_copy(src_hbm.at[wid], src_v, gsem.at[0])
        dst_cp = pltpu.async_copy(dst_hbm.at[wid, pl.ds(0, DSTG)], dst_v,
                                  gsem.at[1])
        src_cp.wait()
        dst_cp.wait()
        init_cp.wait()
        plsc.subcore_barrier()

        def sidx(j):
            return src_v.at[pl.ds(j * CHUNK, CHUNK)]

        def wait_gather(j, b):
            pltpu.make_async_copy(g_hbm.at[sidx(j)], rows_v[b],
                                  gsem.at[b]).wait()

        def start_scatter(j, b, doff):
            pltpu.async_copy(rows_v[b], acc_sh.at[dst_v.at[j - doff]],
                             ssem.at[b], add=True)

        def wait_scatter(j, b, doff):
            pltpu.make_async_copy(rows_v[b], acc_sh.at[dst_v.at[j - doff]],
                                  ssem.at[b]).wait()

        def make_group(start, doff, guard):
            def body(m, _):
                js = [start + 3 * m + t for t in range(3)]
                bs = [(start + t) % 3 for t in range(3)]
                for j, b in zip(js, bs):
                    wait_gather(j, b)
                    start_scatter(j, b, doff)
                for t, (j, b) in enumerate(zip(js, bs)):
                    wait_scatter(j, b, doff)
                    if guard is None or t == 0:
                        pltpu.async_copy(g_hbm.at[sidx(j + 3)], rows_v[b],
                                         gsem.at[b])
                    else:
                        @pl.when(m < guard)
                        def _():
                            pltpu.async_copy(g_hbm.at[sidx(j + 3)],
                                             rows_v[b], gsem.at[b])
                return ()

            return body

        for b in range(NBUF):
            pltpu.async_copy(g_hbm.at[sidx(b)], rows_v[b], gsem.at[b])
        lax.fori_loop(0, NGA, make_group(0, 0, None), ())
        wait_gather(63, 0)
        start_scatter(63, 0, 0)
        wait_scatter(63, 0, 0)
        ref_cp = pltpu.async_copy(dst_hbm.at[wid, pl.ds(DOFF_B, DSTG)],
                                  dst_v, isem)
        pltpu.async_copy(g_hbm.at[sidx(66)], rows_v[0], gsem.at[0])
        ref_cp.wait()
        lax.fori_loop(0, NGB, make_group(64, DOFF_B, NGB - 1), ())
        wait_gather(124, 1)
        start_scatter(124, 1, DOFF_B)
        wait_scatter(124, 1, DOFF_B)
        plsc.subcore_barrier()
        pltpu.sync_copy(acc_sh.at[pl.ds(s * RSTRIDE, RSPAN)],
                        out_hbm.at[pl.ds(c * N + s * RSTRIDE, RSPAN)])

    return deg_kernel, edge_kernel


def _matmul_body(x_ref, w_ref, h_ref):
    h_ref[...] = jnp.dot(x_ref[...], w_ref[...],
                         preferred_element_type=jnp.float32)


_matmul_call = pl.pallas_call(
    _matmul_body,
    out_shape=jax.ShapeDtypeStruct((N, D), jnp.float32),
)


def _scale_body(pT_ref, h_ref, dis_ref, g_ref):
    ones = jnp.ones((NC, D), jnp.float32)
    deg = jnp.dot(pT_ref[...], ones, preferred_element_type=jnp.float32) + 1.0
    dis = lax.rsqrt(deg)
    dis_ref[...] = dis
    g_ref[...] = h_ref[...] * dis


_scale_call = pl.pallas_call(
    _scale_body,
    out_shape=[
        jax.ShapeDtypeStruct((N, D), jnp.float32),
        jax.ShapeDtypeStruct((N, D), jnp.float32),
    ],
)


def _combine_body(acc_ref, g_ref, dis_ref, b_ref, gam_ref, bet_ref, w_ref,
                  x_ref, g_out_ref):
    acc = acc_ref[0:N, :] + acc_ref[N:2 * N, :]
    dis = dis_ref[...]
    pre = dis * (acc - g_ref[...]) + b_ref[...]
    y = jnp.maximum(pre, 0.0)
    m = jnp.mean(y, axis=0, keepdims=True)
    d = y - m
    v = jnp.mean(d * d, axis=0, keepdims=True)
    xn = gam_ref[...] * d * lax.rsqrt(v + EPS) + bet_ref[...]
    x_ref[...] = xn
    g_out_ref[...] = jnp.dot(xn, w_ref[...],
                             preferred_element_type=jnp.float32) * dis


_combine_call = pl.pallas_call(
    _combine_body,
    out_shape=[
        jax.ShapeDtypeStruct((N, D), jnp.float32),
        jax.ShapeDtypeStruct((N, D), jnp.float32),
    ],
)


def _combine_final_body(acc_ref, g_ref, dis_ref, b_ref, gam_ref, bet_ref,
                        x_ref):
    acc = acc_ref[0:N, :] + acc_ref[N:2 * N, :]
    pre = dis_ref[...] * (acc - g_ref[...]) + b_ref[...]
    y = jnp.maximum(pre, 0.0)
    m = jnp.mean(y, axis=0, keepdims=True)
    d = y - m
    v = jnp.mean(d * d, axis=0, keepdims=True)
    x_ref[...] = gam_ref[...] * d * lax.rsqrt(v + EPS) + bet_ref[...]


_combine_final_call = pl.pallas_call(
    _combine_final_body,
    out_shape=jax.ShapeDtypeStruct((N, D), jnp.float32),
)


def kernel(edge_index, node_attr, edge_attr,
           W1, b1, g1, be1, W2, b2, g2, be2, W3, b3, g3, be3):
    del edge_attr
    src = edge_index[0].astype(jnp.int32).reshape(NW, EPW)
    dst = edge_index[1].astype(jnp.int32).reshape(NW, NCH, CHUNK)
    dst_pad = jnp.pad(dst, ((0, 0), (0, 128 - NCH), (0, 0)))
    b1r = b1.reshape(1, D); b2r = b2.reshape(1, D); b3r = b3.reshape(1, D)
    g1r = g1.reshape(1, D); g2r = g2.reshape(1, D); g3r = g3.reshape(1, D)
    be1r = be1.reshape(1, D); be2r = be2.reshape(1, D); be3r = be3.reshape(1, D)

    deg_kernel, edge_kernel = _sc_kernels()
    partials = deg_kernel(dst)
    h1 = _matmul_call(node_attr, W1)
    pT = partials.reshape(NC, NPAD)[:, :N].T
    dis, gg = _scale_call(pT, h1)

    acc = edge_kernel(gg, src, dst_pad)
    x2, gg2 = _combine_call(acc, gg, dis, b1r, g1r, be1r, W2)
    del x2
    acc2 = edge_kernel(gg2, src, dst_pad)
    x3, gg3 = _combine_call(acc2, gg2, dis, b2r, g2r, be2r, W3)
    del x3
    acc3 = edge_kernel(gg3, src, dst_pad)
    return _combine_final_call(acc3, gg3, dis, b3r, g3r, be3r)


# speedup vs baseline: 26.9267x; 1.0312x over previous
import functools

import jax
import jax.numpy as jnp
from jax import lax
from jax.experimental import pallas as pl
from jax.experimental.pallas import tpu as pltpu
from jax.experimental.pallas import tpu_sc as plsc

N = 10000
D = 128
E = 320000
NC = 2
NS = 16
NW = NC * NS
EPW = E // NW
CHUNK = 80
NCH = EPW // CHUNK
RSTRIDE = 624
RSPAN = 640
NPAD = 10240
PPT = NPAD // NS
EPS = 1e-5

@functools.cache
def _sc_kernels():
    mesh = plsc.VectorSubcoreMesh(core_axis_name="c", subcore_axis_name="s")

    @functools.partial(
        pl.kernel,
        mesh=mesh,
        out_type=jax.ShapeDtypeStruct((NC * NPAD,), jnp.float32),
        scratch_types=[
            pltpu.VMEM((NCH, CHUNK), jnp.int32),
            pltpu.VMEM((PPT,), jnp.float32),
            pltpu.VMEM_SHARED((NPAD,), jnp.float32),
            pltpu.SemaphoreType.DMA,
        ],
    )
    def deg_kernel(dst_hbm, out_hbm, dst_v, zv, deg_sh, dsem):
        c = lax.axis_index("c")
        s = lax.axis_index("s")
        wid = s * NC + c

        def zero_body(i, _):
            zv[pl.ds(i * 16, 16)] = jnp.zeros((16,), jnp.float32)
            return ()

        lax.fori_loop(0, PPT // 16, zero_body, ())
        pltpu.sync_copy(zv, deg_sh.at[pl.ds(s * PPT, PPT)])
        pltpu.sync_copy(dst_hbm.at[wid], dst_v)

        def ones_body(i, _):
            zv[pl.ds(i * 16, 16)] = jnp.ones((16,), jnp.float32)
            return ()

        lax.fori_loop(0, CHUNK // 16, ones_body, ())
        plsc.subcore_barrier()

        ones_ref = zv.at[pl.ds(0, CHUNK)]

        def body(gidx, _):
            for t in range(5):
                pltpu.async_copy(ones_ref, deg_sh.at[dst_v.at[5 * gidx + t]],
                                 dsem, add=True)
            for t in range(5):
                pltpu.make_async_copy(
                    ones_ref, deg_sh.at[dst_v.at[5 * gidx + t]], dsem).wait()
            return ()

        lax.fori_loop(0, NCH // 5, body, ())
        plsc.subcore_barrier()
        pltpu.sync_copy(deg_sh.at[pl.ds(s * PPT, PPT)],
                        out_hbm.at[pl.ds(c * NPAD + s * PPT, PPT)])

    NBUF = 3
    DSTG = 64
    NGA = 21
    NGB = 20
    DOFF_B = 64

    @functools.partial(
        pl.kernel,
        mesh=mesh,
        out_type=jax.ShapeDtypeStruct((NC * N, D), jnp.float32),
        scratch_types=[
            pltpu.VMEM((EPW,), jnp.int32),
            pltpu.VMEM((DSTG, CHUNK), jnp.int32),
            pltpu.VMEM((NBUF, CHUNK, D), jnp.float32),
            pltpu.VMEM_SHARED((N, D), jnp.float32),
            pltpu.SemaphoreType.DMA((NBUF,)),
            pltpu.SemaphoreType.DMA((NBUF,)),
            pltpu.SemaphoreType.DMA,
        ],
    )
    def edge_kernel(g_hbm, src_hbm, dst_hbm, out_hbm, src_v, dst_v, rows_b,
                    acc_sh, gsem, ssem, isem):
        rows_v = [rows_b.at[b] for b in range(NBUF)]
        c = lax.axis_index("c")
        s = lax.axis_index("s")
        wid = s * NC + c
        init_cp = pltpu.async_copy(
            g_hbm.at[pl.ds(s * RSTRIDE, RSPAN)],
            acc_sh.at[pl.ds(s * RSTRIDE, RSPAN)], isem)
        src_cp = pltpu.async_copy(src_hbm.at[wid], src_v, gsem.at[0])
        dst_cp = pltpu.async_copy(dst_hbm.at[wid, pl.ds(0, DSTG)], dst_v,
                                  isem)
        src_cp.wait()

        def sidx(j):
            return src_v.at[pl.ds(j * CHUNK, CHUNK)]

        for b in range(NBUF):
            pltpu.async_copy(g_hbm.at[sidx(b)], rows_v[b], gsem.at[b])
        dst_cp.wait()
        init_cp.wait()
        plsc.subcore_barrier()

        def wait_gather(j, b):
            pltpu.make_async_copy(g_hbm.at[sidx(j)], rows_v[b],
                                  gsem.at[b]).wait()

        def start_scatter(j, b, doff):
            pltpu.async_copy(rows_v[b], acc_sh.at[dst_v.at[j - doff]],
                             ssem.at[b], add=True)

        def wait_scatter(j, b, doff):
            pltpu.make_async_copy(rows_v[b], acc_sh.at[dst_v.at[j - doff]],
                                  ssem.at[b]).wait()

        def make_group(start, doff, guard):
            def body(m, _):
                js = [start + 3 * m + t for t in range(3)]
                bs = [(start + t) % 3 for t in range(3)]
                for j, b in zip(js, bs):
                    wait_gather(j, b)
                    start_scatter(j, b, doff)
                for t, (j, b) in enumerate(zip(js, bs)):
                    wait_scatter(j, b, doff)
                    if guard is None or t == 0:
                        pltpu.async_copy(g_hbm.at[sidx(j + 3)], rows_v[b],
                                         gsem.at[b])
                    else:
                        @pl.when(m < guard)
                        def _():
                            pltpu.async_copy(g_hbm.at[sidx(j + 3)],
                                             rows_v[b], gsem.at[b])
                return ()

            return body

        lax.fori_loop(0, NGA, make_group(0, 0, None), ())
        wait_gather(63, 0)
        start_scatter(63, 0, 0)
        wait_scatter(63, 0, 0)
        ref_cp = pltpu.async_copy(dst_hbm.at[wid, pl.ds(DOFF_B, DSTG)],
                                  dst_v, isem)
        pltpu.async_copy(g_hbm.at[sidx(66)], rows_v[0], gsem.at[0])
        ref_cp.wait()
        lax.fori_loop(0, NGB, make_group(64, DOFF_B, NGB - 1), ())
        wait_gather(124, 1)
        start_scatter(124, 1, DOFF_B)
        wait_scatter(124, 1, DOFF_B)
        plsc.subcore_barrier()
        pltpu.sync_copy(acc_sh.at[pl.ds(s * RSTRIDE, RSPAN)],
                        out_hbm.at[pl.ds(c * N + s * RSTRIDE, RSPAN)])

    return deg_kernel, edge_kernel


def _prep_body(pT_ref, x_ref, w_ref, dis_ref, g_ref):
    ones = jnp.ones((NC, D), jnp.float32)
    deg = jnp.dot(pT_ref[...], ones, preferred_element_type=jnp.float32) + 1.0
    dis = lax.rsqrt(deg)
    h = jnp.dot(x_ref[...], w_ref[...], preferred_element_type=jnp.float32)
    dis_ref[...] = dis
    g_ref[...] = h * dis


_prep_call = pl.pallas_call(
    _prep_body,
    out_shape=[
        jax.ShapeDtypeStruct((N, D), jnp.float32),
        jax.ShapeDtypeStruct((N, D), jnp.float32),
    ],
)


def _combine_body(acc_ref, g_ref, dis_ref, b_ref, gam_ref, bet_ref, w_ref,
                  x_ref, g_out_ref):
    acc = acc_ref[0:N, :] + acc_ref[N:2 * N, :]
    dis = dis_ref[...]
    pre = dis * (acc - g_ref[...]) + b_ref[...]
    y = jnp.maximum(pre, 0.0)
    m = jnp.mean(y, axis=0, keepdims=True)
    d = y - m
    v = jnp.mean(d * d, axis=0, keepdims=True)
    xn = gam_ref[...] * d * lax.rsqrt(v + EPS) + bet_ref[...]
    x_ref[...] = xn
    g_out_ref[...] = jnp.dot(xn, w_ref[...],
                             preferred_element_type=jnp.float32) * dis


_combine_call = pl.pallas_call(
    _combine_body,
    out_shape=[
        jax.ShapeDtypeStruct((N, D), jnp.float32),
        jax.ShapeDtypeStruct((N, D), jnp.float32),
    ],
)


def _combine_final_body(acc_ref, g_ref, dis_ref, b_ref, gam_ref, bet_ref,
                        x_ref):
    acc = acc_ref[0:N, :] + acc_ref[N:2 * N, :]
    pre = dis_ref[...] * (acc - g_ref[...]) + b_ref[...]
    y = jnp.maximum(pre, 0.0)
    m = jnp.mean(y, axis=0, keepdims=True)
    d = y - m
    v = jnp.mean(d * d, axis=0, keepdims=True)
    x_ref[...] = gam_ref[...] * d * lax.rsqrt(v + EPS) + bet_ref[...]


_combine_final_call = pl.pallas_call(
    _combine_final_body,
    out_shape=jax.ShapeDtypeStruct((N, D), jnp.float32),
)


def kernel(edge_index, node_attr, edge_attr,
           W1, b1, g1, be1, W2, b2, g2, be2, W3, b3, g3, be3):
    del edge_attr
    src = edge_index[0].astype(jnp.int32).reshape(NW, EPW)
    dst = edge_index[1].astype(jnp.int32).reshape(NW, NCH, CHUNK)
    dst_pad = jnp.pad(dst, ((0, 0), (0, 128 - NCH), (0, 0)))
    b1r = b1.reshape(1, D); b2r = b2.reshape(1, D); b3r = b3.reshape(1, D)
    g1r = g1.reshape(1, D); g2r = g2.reshape(1, D); g3r = g3.reshape(1, D)
    be1r = be1.reshape(1, D); be2r = be2.reshape(1, D); be3r = be3.reshape(1, D)

    deg_kernel, edge_kernel = _sc_kernels()
    partials = deg_kernel(dst)
    pT = partials.reshape(NC, NPAD)[:, :N].T
    dis, gg = _prep_call(pT, node_attr, W1)

    acc = edge_kernel(gg, src, dst_pad)
    x2, gg2 = _combine_call(acc, gg, dis, b1r, g1r, be1r, W2)
    del x2
    acc2 = edge_kernel(gg2, src, dst_pad)
    x3, gg3 = _combine_call(acc2, gg2, dis, b2r, g2r, be2r, W3)
    del x3
    acc3 = edge_kernel(gg3, src, dst_pad)
    return _combine_final_call(acc3, gg3, dis, b3r, g3r, be3r)
